# baseline (speedup 1.0000x reference)
import functools

import jax
import jax.numpy as jnp
from jax import lax
from jax.experimental import pallas as pl
from jax.experimental.pallas import tpu as pltpu
from jax.experimental.pallas import tpu_sc as plsc

DIM = 64
BATCH = 16384
HALF = BATCH // 2
MARGIN = 1.0

NW = 32
BPW = BATCH // NW
CHUNK = 64
NCHUNK = BPW // CHUNK
NBUF = 2
L = 16

_N_ENT = 1_000_000
_NROW = _N_ENT // 2

TBLK = 4096


def _tp_body(x_ref, o_ref):
    x = x_ref[...]
    row = lax.broadcasted_iota(jnp.int32, (DIM, DIM), 0)
    col = lax.broadcasted_iota(jnp.int32, (DIM, DIM), 1)
    eye = (row == col).astype(jnp.float32)
    o_ref[...] = lax.dot_general(
        x, eye, (((0,), (0,)), ((), ())), preferred_element_type=jnp.float32
    )


_tc_transpose = pl.pallas_call(
    _tp_body,
    grid=(pl.cdiv(_N_ENT, TBLK),),
    in_specs=[pl.BlockSpec((DIM, TBLK), lambda i: (0, i))],
    out_specs=pl.BlockSpec((TBLK, DIM), lambda i: (i, 0)),
    out_shape=jax.ShapeDtypeStruct((_N_ENT, DIM), jnp.float32),
)


def _sc_scores(h_idx, t_idx, r_idx, ent_re2, ent_im2, rel_re2, rel_im2):
    mesh = plsc.VectorSubcoreMesh(core_axis_name="c", subcore_axis_name="s")
    row_buf = lambda: pltpu.VMEM((CHUNK, 2 * DIM), jnp.float32)

    @functools.partial(
        pl.kernel,
        mesh=mesh,
        compiler_params=pltpu.CompilerParams(
            needs_layout_passes=False, use_tc_tiling_on_sc=True
        ),
        out_type=jax.ShapeDtypeStruct((BATCH,), jnp.float32),
        scratch_types=(
            [pltpu.VMEM((BPW,), jnp.int32) for _ in range(3)]
            + [pltpu.VMEM((BPW,), jnp.int32) for _ in range(3)]
            + [pltpu.VMEM((BPW,), jnp.int32) for _ in range(3)]
            + [row_buf() for _ in range(6 * NBUF)]
            + [pltpu.VMEM((BPW,), jnp.float32)]
            + [pltpu.SemaphoreType.DMA for _ in range(NBUF)]
        ),
    )
    def k(h_hbm, t_hbm, r_hbm, ere_hbm, eim_hbm, rre_hbm, rim_hbm, out_hbm,
          ih, it, ir, ih2, it2, ir2, hp, tp, rp, *rest):
        bufs = [rest[6 * s:6 * (s + 1)] for s in range(NBUF)]
        score = rest[6 * NBUF]
        sems = rest[6 * NBUF + 1:]
        wid = lax.axis_index("s") * 2 + lax.axis_index("c")
        base = wid * BPW
        pltpu.sync_copy(h_hbm.at[pl.ds(base, BPW)], ih)
        pltpu.sync_copy(t_hbm.at[pl.ds(base, BPW)], it)
        pltpu.sync_copy(r_hbm.at[pl.ds(base, BPW)], ir)

        def split(j, _):
            sl = pl.ds(j * L, L)
            for src, half, par in ((ih, ih2, hp), (it, it2, tp), (ir, ir2, rp)):
                v = src[sl]
                half[sl] = lax.shift_right_logical(v, 1)
                par[sl] = lax.shift_left((v & 1), 6)
            return 0

        lax.fori_loop(0, BPW // L, split, 0)

        def start(c):
            s = c % NBUF
            hre, him, tre, tim, rre, rim = bufs[s]
            hh = ih2.at[pl.ds(c * CHUNK, CHUNK)]
            tt = it2.at[pl.ds(c * CHUNK, CHUNK)]
            rr = ir2.at[pl.ds(c * CHUNK, CHUNK)]
            return [
                pltpu.async_copy(ere_hbm.at[hh], hre, sems[s]),
                pltpu.async_copy(eim_hbm.at[hh], him, sems[s]),
                pltpu.async_copy(ere_hbm.at[tt], tre, sems[s]),
                pltpu.async_copy(eim_hbm.at[tt], tim, sems[s]),
                pltpu.async_copy(rre_hbm.at[rr], rre, sems[s]),
                pltpu.async_copy(rim_hbm.at[rr], rim, sems[s]),
            ]

        iota = lax.iota(jnp.int32, L)

        def compute(c):
            hre, him, tre, tim, rre, rim = bufs[c % NBUF]
            for g in range(CHUNK // L):
                rows = iota + (g * L)
                sl = pl.ds(c * CHUNK + g * L, L)
                hp16 = hp[sl]
                tp16 = tp[sl]
                rp16 = rp[sl]

                def body(d, acc):
                    ch = hp16 + d
                    ct = tp16 + d
                    cr = rp16 + d
                    xhre = plsc.load_gather(hre, [rows, ch])
                    xhim = plsc.load_gather(him, [rows, ch])
                    xtre = plsc.load_gather(tre, [rows, ct])
                    xtim = plsc.load_gather(tim, [rows, ct])
                    xrre = plsc.load_gather(rre, [rows, cr])
                    xrim = plsc.load_gather(rim, [rows, cr])
                    re_part = xhre * xtre + xhim * xtim
                    im_part = xhre * xtim - xhim * xtre
                    return acc + (xrre * re_part + xrim * im_part)

                acc = lax.fori_loop(0, DIM, body, jnp.zeros((L,), jnp.float32))
                score[sl] = -acc

        handles = start(0)
        for c in range(NCHUNK):
            nxt = start(c + 1) if c + 1 < NCHUNK else None
            for hnd in handles:
                hnd.wait()
            compute(c)
            handles = nxt
        pltpu.sync_copy(score, out_hbm.at[pl.ds(base, BPW)])

    return k(h_idx, t_idx, r_idx, ent_re2, ent_im2, rel_re2, rel_im2)


def _loss_body(s_ref, out_ref):
    s = s_ref[...]
    pos = s[:HALF // 128, :]
    neg = s[HALF // 128:, :]
    out_ref[0, 0] = jnp.sum(jnp.maximum(pos - neg + MARGIN, 0.0))


_tc_loss = pl.pallas_call(
    _loss_body,
    out_shape=jax.ShapeDtypeStruct((1, 1), jnp.float32),
    out_specs=pl.BlockSpec(memory_space=pltpu.SMEM),
)


def kernel(batch_h, batch_t, batch_r, batch_y, ent_re, ent_im, rel_re, rel_im):
    del batch_y
    h = batch_h.astype(jnp.int32)
    t = batch_t.astype(jnp.int32)
    r = batch_r.astype(jnp.int32)
    e_re2 = ent_re.reshape(_NROW, 2 * DIM)
    e_im2 = ent_im.reshape(_NROW, 2 * DIM)
    r_re2 = rel_re.reshape(_NROW, 2 * DIM)
    r_im2 = _tc_transpose(rel_im.T).reshape(_NROW, 2 * DIM)
    probe = (e_re2[0, :8] + e_im2[0, :8] + r_re2[0, :8] + r_im2[0, :8])
    zero = (jnp.min(jnp.abs(probe)) * 0.0).astype(jnp.int32)
    score = _sc_scores(h + zero, t + zero, r + zero, e_re2, e_im2, r_re2, r_im2)
    loss = _tc_loss(score.reshape(BATCH // 128, 128))[0, 0]
    return (loss, score[:HALF], score[HALF:])


# speedup vs baseline: 1.0846x; 1.0846x over previous
import functools

import jax
import jax.numpy as jnp
from jax import lax
from jax.experimental import pallas as pl
from jax.experimental.pallas import tpu as pltpu
from jax.experimental.pallas import tpu_sc as plsc

DIM = 64
BATCH = 16384
HALF = BATCH // 2
MARGIN = 1.0

NW = 32
BPW = BATCH // NW
CHUNK = 64
NCHUNK = BPW // CHUNK
NBUF = 2
L = 16

_N_ENT = 1_000_000
_NROW = _N_ENT // 2

TBLK = 4096


def _tp_body(x_ref, o_ref):
    x = x_ref[...]
    row = lax.broadcasted_iota(jnp.int32, (DIM, DIM), 0)
    col = lax.broadcasted_iota(jnp.int32, (DIM, DIM), 1)
    eye = (row == col).astype(jnp.float32)
    o_ref[...] = lax.dot_general(
        x, eye, (((0,), (0,)), ((), ())), preferred_element_type=jnp.float32
    )


_tc_transpose = pl.pallas_call(
    _tp_body,
    grid=(pl.cdiv(_N_ENT, TBLK),),
    in_specs=[pl.BlockSpec((DIM, TBLK), lambda i: (0, i))],
    out_specs=pl.BlockSpec((TBLK, DIM), lambda i: (i, 0)),
    out_shape=jax.ShapeDtypeStruct((_N_ENT, DIM), jnp.float32),
)


def _sc_scores(h_idx, t_idx, r_idx, ent_re2, ent_im2, rel_re2, rel_im2):
    mesh = plsc.VectorSubcoreMesh(core_axis_name="c", subcore_axis_name="s")
    row_buf = lambda: pltpu.VMEM((CHUNK, 2 * DIM), jnp.float32)

    @functools.partial(
        pl.kernel,
        mesh=mesh,
        compiler_params=pltpu.CompilerParams(
            needs_layout_passes=False, use_tc_tiling_on_sc=True
        ),
        out_type=jax.ShapeDtypeStruct((BATCH,), jnp.float32),
        scratch_types=(
            [pltpu.VMEM((BPW,), jnp.int32) for _ in range(3)]
            + [pltpu.VMEM((BPW,), jnp.int32) for _ in range(3)]
            + [pltpu.VMEM((BPW,), jnp.int32) for _ in range(3)]
            + [row_buf() for _ in range(6 * NBUF)]
            + [pltpu.VMEM((BPW,), jnp.float32)]
            + [pltpu.SemaphoreType.DMA for _ in range(NBUF)]
        ),
    )
    def k(h_hbm, t_hbm, r_hbm, ere_hbm, eim_hbm, rre_hbm, rim_hbm, out_hbm,
          ih, it, ir, ih2, it2, ir2, hp, tp, rp, *rest):
        bufs = [rest[6 * s:6 * (s + 1)] for s in range(NBUF)]
        score = rest[6 * NBUF]
        sems = rest[6 * NBUF + 1:]
        wid = lax.axis_index("s") * 2 + lax.axis_index("c")
        base = wid * BPW
        pltpu.sync_copy(h_hbm.at[pl.ds(base, BPW)], ih)
        pltpu.sync_copy(t_hbm.at[pl.ds(base, BPW)], it)
        pltpu.sync_copy(r_hbm.at[pl.ds(base, BPW)], ir)

        def split(j, _):
            sl = pl.ds(j * L, L)
            for src, half, par in ((ih, ih2, hp), (it, it2, tp), (ir, ir2, rp)):
                v = src[sl]
                half[sl] = lax.shift_right_logical(v, 1)
                par[sl] = lax.shift_left((v & 1), 6)
            return 0

        lax.fori_loop(0, BPW // L, split, 0)

        def start(c):
            s = c % NBUF
            hre, him, tre, tim, rre, rim = bufs[s]
            hh = ih2.at[pl.ds(c * CHUNK, CHUNK)]
            tt = it2.at[pl.ds(c * CHUNK, CHUNK)]
            rr = ir2.at[pl.ds(c * CHUNK, CHUNK)]
            return [
                pltpu.async_copy(ere_hbm.at[hh], hre, sems[s]),
                pltpu.async_copy(eim_hbm.at[hh], him, sems[s]),
                pltpu.async_copy(ere_hbm.at[tt], tre, sems[s]),
                pltpu.async_copy(eim_hbm.at[tt], tim, sems[s]),
                pltpu.async_copy(rre_hbm.at[rr], rre, sems[s]),
                pltpu.async_copy(rim_hbm.at[rr], rim, sems[s]),
            ]

        iota = lax.iota(jnp.int32, L)

        def compute(c):
            hre, him, tre, tim, rre, rim = bufs[c % NBUF]
            for g in range(CHUNK // L):
                rows = iota + (g * L)
                sl = pl.ds(c * CHUNK + g * L, L)
                hp16 = hp[sl]
                tp16 = tp[sl]
                rp16 = rp[sl]

                def body(d, acc):
                    ch = hp16 + d
                    ct = tp16 + d
                    cr = rp16 + d
                    xhre = plsc.load_gather(hre, [rows, ch])
                    xhim = plsc.load_gather(him, [rows, ch])
                    xtre = plsc.load_gather(tre, [rows, ct])
                    xtim = plsc.load_gather(tim, [rows, ct])
                    xrre = plsc.load_gather(rre, [rows, cr])
                    xrim = plsc.load_gather(rim, [rows, cr])
                    re_part = xhre * xtre + xhim * xtim
                    im_part = xhre * xtim - xhim * xtre
                    return acc + (xrre * re_part + xrim * im_part)

                acc = lax.fori_loop(0, DIM, body, jnp.zeros((L,), jnp.float32))
                score[sl] = -acc

        handles = start(0)
        for c in range(NCHUNK):
            nxt = start(c + 1) if c + 1 < NCHUNK else None
            for hnd in handles:
                hnd.wait()
            compute(c)
            handles = nxt
        pltpu.sync_copy(score, out_hbm.at[pl.ds(base, BPW)])

    return k(h_idx, t_idx, r_idx, ent_re2, ent_im2, rel_re2, rel_im2)


def _loss_body(s_ref, out_ref):
    s = s_ref[...]
    pos = s[:HALF // 128, :]
    neg = s[HALF // 128:, :]
    out_ref[0, 0] = jnp.sum(jnp.maximum(pos - neg + MARGIN, 0.0))


_tc_loss = pl.pallas_call(
    _loss_body,
    out_shape=jax.ShapeDtypeStruct((1, 1), jnp.float32),
    out_specs=pl.BlockSpec(memory_space=pltpu.SMEM),
)


def kernel(batch_h, batch_t, batch_r, batch_y, ent_re, ent_im, rel_re, rel_im):
    del batch_y
    h = batch_h.astype(jnp.int32)
    t = batch_t.astype(jnp.int32)
    r = batch_r.astype(jnp.int32)
    e_re2, e_im2, r_re2, r_im2 = lax.optimization_barrier((
        ent_re.reshape(_NROW, 2 * DIM),
        ent_im.reshape(_NROW, 2 * DIM),
        rel_re.reshape(_NROW, 2 * DIM),
        rel_im.reshape(_NROW, 2 * DIM),
    ))
    score = _sc_scores(h, t, r, e_re2, e_im2, r_re2, r_im2)
    loss = _tc_loss(score.reshape(BATCH // 128, 128))[0, 0]
    return (loss, score[:HALF], score[HALF:])


# speedup vs baseline: 1.0849x; 1.0003x over previous
import functools

import jax
import jax.numpy as jnp
from jax import lax
from jax.experimental import pallas as pl
from jax.experimental.pallas import tpu as pltpu
from jax.experimental.pallas import tpu_sc as plsc

DIM = 64
BATCH = 16384
HALF = BATCH // 2
MARGIN = 1.0

NW = 32
BPW = BATCH // NW
CHUNK = 64
NCHUNK = BPW // CHUNK
NBUF = 2
L = 16

_N_ENT = 1_000_000
_NROW = _N_ENT // 2


def _sc_scores(h_idx, t_idx, r_idx, ent_re2, ent_im2, rel_re2, rel_im2):
    mesh = plsc.VectorSubcoreMesh(core_axis_name="c", subcore_axis_name="s")
    row_buf = lambda: pltpu.VMEM((CHUNK, 2 * DIM), jnp.float32)

    @functools.partial(
        pl.kernel,
        mesh=mesh,
        compiler_params=pltpu.CompilerParams(
            needs_layout_passes=False, use_tc_tiling_on_sc=True
        ),
        out_type=jax.ShapeDtypeStruct((BATCH,), jnp.float32),
        scratch_types=(
            [pltpu.VMEM((BPW,), jnp.int32) for _ in range(3)]
            + [pltpu.VMEM((BPW,), jnp.int32) for _ in range(3)]
            + [pltpu.VMEM((BPW,), jnp.int32) for _ in range(3)]
            + [row_buf() for _ in range(6 * NBUF)]
            + [pltpu.VMEM((BPW,), jnp.float32)]
            + [pltpu.SemaphoreType.DMA for _ in range(NBUF)]
        ),
    )
    def k(h_hbm, t_hbm, r_hbm, ere_hbm, eim_hbm, rre_hbm, rim_hbm, out_hbm,
          ih, it, ir, ih2, it2, ir2, hp, tp, rp, *rest):
        bufs = [rest[6 * s:6 * (s + 1)] for s in range(NBUF)]
        score = rest[6 * NBUF]
        sems = rest[6 * NBUF + 1:]
        wid = lax.axis_index("s") * 2 + lax.axis_index("c")
        base = wid * BPW
        pltpu.sync_copy(h_hbm.at[pl.ds(base, BPW)], ih)
        pltpu.sync_copy(t_hbm.at[pl.ds(base, BPW)], it)
        pltpu.sync_copy(r_hbm.at[pl.ds(base, BPW)], ir)

        def split(j, _):
            sl = pl.ds(j * L, L)
            for src, half, par in ((ih, ih2, hp), (it, it2, tp), (ir, ir2, rp)):
                v = src[sl]
                half[sl] = lax.shift_right_logical(v, 1)
                par[sl] = lax.shift_left((v & 1), 6)
            return 0

        lax.fori_loop(0, BPW // L, split, 0)

        def start(c):
            s = c % NBUF
            hre, him, tre, tim, rre, rim = bufs[s]
            hh = ih2.at[pl.ds(c * CHUNK, CHUNK)]
            tt = it2.at[pl.ds(c * CHUNK, CHUNK)]
            rr = ir2.at[pl.ds(c * CHUNK, CHUNK)]
            return [
                pltpu.async_copy(ere_hbm.at[hh], hre, sems[s]),
                pltpu.async_copy(eim_hbm.at[hh], him, sems[s]),
                pltpu.async_copy(ere_hbm.at[tt], tre, sems[s]),
                pltpu.async_copy(eim_hbm.at[tt], tim, sems[s]),
                pltpu.async_copy(rre_hbm.at[rr], rre, sems[s]),
                pltpu.async_copy(rim_hbm.at[rr], rim, sems[s]),
            ]

        iota = lax.iota(jnp.int32, L)

        def compute(c):
            hre, him, tre, tim, rre, rim = bufs[c % NBUF]
            for g in range(CHUNK // L):
                rows = iota + (g * L)
                sl = pl.ds(c * CHUNK + g * L, L)
                hp16 = hp[sl]
                tp16 = tp[sl]
                rp16 = rp[sl]

                def body(d, acc):
                    ch = hp16 + d
                    ct = tp16 + d
                    cr = rp16 + d
                    xhre = plsc.load_gather(hre, [rows, ch])
                    xhim = plsc.load_gather(him, [rows, ch])
                    xtre = plsc.load_gather(tre, [rows, ct])
                    xtim = plsc.load_gather(tim, [rows, ct])
                    xrre = plsc.load_gather(rre, [rows, cr])
                    xrim = plsc.load_gather(rim, [rows, cr])
                    re_part = xhre * xtre + xhim * xtim
                    im_part = xhre * xtim - xhim * xtre
                    return acc + (xrre * re_part + xrim * im_part)

                acc = lax.fori_loop(0, DIM, body, jnp.zeros((L,), jnp.float32))
                score[sl] = -acc

        handles = start(0)
        for c in range(NCHUNK):
            nxt = start(c + 1) if c + 1 < NCHUNK else None
            for hnd in handles:
                hnd.wait()
            compute(c)
            handles = nxt
        pltpu.sync_copy(score, out_hbm.at[pl.ds(base, BPW)])

    return k(h_idx, t_idx, r_idx, ent_re2, ent_im2, rel_re2, rel_im2)


def _loss_body(s_ref, out_ref):
    s = s_ref[...]
    pos = s[:HALF // 128, :]
    neg = s[HALF // 128:, :]
    out_ref[0, 0] = jnp.sum(jnp.maximum(pos - neg + MARGIN, 0.0))


_tc_loss = pl.pallas_call(
    _loss_body,
    out_shape=jax.ShapeDtypeStruct((1, 1), jnp.float32),
    out_specs=pl.BlockSpec(memory_space=pltpu.SMEM),
)


def kernel(batch_h, batch_t, batch_r, batch_y, ent_re, ent_im, rel_re, rel_im):
    del batch_y
    h = batch_h.astype(jnp.int32)
    t = batch_t.astype(jnp.int32)
    r = batch_r.astype(jnp.int32)
    e_re2, e_im2, r_re2, r_im2 = lax.optimization_barrier((
        ent_re.reshape(_NROW, 2 * DIM),
        ent_im.reshape(_NROW, 2 * DIM),
        rel_re.reshape(_NROW, 2 * DIM),
        rel_im.reshape(_NROW, 2 * DIM),
    ))
    score = _sc_scores(h, t, r, e_re2, e_im2, r_re2, r_im2)
    loss = _tc_loss(score.reshape(BATCH // 128, 128))[0, 0]
    return (loss, score[:HALF], score[HALF:])
